# baseline (speedup 1.0000x reference)
import functools

import jax
import jax.numpy as jnp
from jax import lax
from jax.experimental import pallas as pl
from jax.experimental.pallas import tpu as pltpu
from jax.experimental.pallas import tpu_sc as plsc

_EMBED_DIM = 128
_FINAL_DIM = 128
_QUERY_LEN = 200

_L = 16
_NV = _EMBED_DIM // _L
_NW = 16
_CPW = 16
_KPW = _FINAL_DIM // _NW

_mesh = plsc.VectorSubcoreMesh(core_axis_name="c", subcore_axis_name="s",
                               num_cores=1)


@functools.partial(
    pl.kernel,
    mesh=_mesh,
    out_type=jax.ShapeDtypeStruct((_FINAL_DIM,), jnp.float32),
    scratch_types=[
        pltpu.VMEM((_CPW,), jnp.int32),
        pltpu.VMEM((_CPW, _EMBED_DIM), jnp.float32),
        pltpu.VMEM((_KPW, _EMBED_DIM), jnp.float32),
        pltpu.VMEM((_EMBED_DIM,), jnp.float32),
        pltpu.VMEM((_NW, _EMBED_DIM), jnp.float32),
        pltpu.VMEM((_L,), jnp.float32),
        pltpu.VMEM_SHARED((_NW, _EMBED_DIM), jnp.float32),
        pltpu.SemaphoreType.DMA,
        pltpu.SemaphoreType.DMA,
    ],
)
def _qi_kernel(query_hbm, table_hbm, w_hbm, out_hbm,
               idx_v, rows_v, w_rows, svec_v, sall_v, out_buf,
               shared, sem_w, sem_g):
    wid = lax.axis_index("s")
    zero = jnp.zeros((_L,), jnp.float32)
    lanes = lax.iota(jnp.int32, _L)

    cp_w = pltpu.async_copy(w_hbm.at[pl.ds(wid * _KPW, _KPW)], w_rows, sem_w)

    pltpu.sync_copy(query_hbm.at[wid], idx_v)
    cp_g = pltpu.async_copy(table_hbm.at[idx_v], rows_v, sem_g)

    cp_g.wait()
    accs = [zero] * _NV
    for c in range(_CPW):
        for j in range(_NV):
            accs[j] = accs[j] + rows_v[c, pl.ds(j * _L, _L)]

    corr = jnp.where(wid == 12, -8.0,
                     jnp.where(wid > 12, jnp.float32(-_CPW), 0.0))
    for j in range(_NV):
        accs[j] = accs[j] + corr * rows_v[_CPW - 1, pl.ds(j * _L, _L)]
        svec_v[pl.ds(j * _L, _L)] = accs[j]

    pltpu.sync_copy(svec_v, shared.at[wid])
    plsc.subcore_barrier()
    pltpu.sync_copy(shared, sall_v)

    s = [zero] * _NV
    for r in range(_NW):
        for j in range(_NV):
            s[j] = s[j] + sall_v[r, pl.ds(j * _L, _L)]

    cp_w.wait()
    outv = zero
    for k in range(_KPW):
        p = s[0] * w_rows[k, pl.ds(0, _L)]
        for j in range(1, _NV):
            p = p + s[j] * w_rows[k, pl.ds(j * _L, _L)]
        for sh in (8, 4, 2, 1):
            p = p + p.at[lanes ^ sh].get(mode="promise_in_bounds")
        outv = jnp.where(lanes == k, p, outv)
    out_buf[...] = outv

    pltpu.sync_copy(out_buf.at[pl.ds(0, _KPW)],
                    out_hbm.at[pl.ds(wid * _KPW, _KPW)])


def kernel(query, table, W):
    q = jnp.pad(query.astype(jnp.int32), (0, _NW * _CPW - _QUERY_LEN))
    return _qi_kernel(q.reshape(_NW, _CPW), table, W)
